# baseline (speedup 1.0000x reference)
import functools

import jax
import jax.numpy as jnp
from jax.experimental import pallas as pl
from jax.experimental.pallas import tpu as pltpu


def _make_body(n, m_blk, depth):
    n_blk = n // m_blk

    def _body(adj_hbm, x_ref, w_ref, out_ref, buf, sems):
        def cp(i, slot):
            return pltpu.make_async_copy(
                adj_hbm.at[pl.ds(i * m_blk, m_blk), :],
                buf.at[slot],
                sems.at[slot],
            )

        for s in range(depth - 1):
            cp(s, s).start()

        def step(i, carry):
            nxt = i + depth - 1

            @pl.when(nxt < n_blk)
            def _start_next():
                cp(nxt, jax.lax.rem(nxt, depth)).start()

            slot = jax.lax.rem(i, depth)
            cp(i, slot).wait()
            h = jnp.dot(buf[slot], x_ref[...],
                        preferred_element_type=jnp.float32)
            out_ref[pl.ds(i * m_blk, m_blk), :] = jax.lax.dot_general(
                h, w_ref[...], (((1,), (1,)), ((), ())),
                preferred_element_type=jnp.float32,
            )
            return carry

        jax.lax.fori_loop(0, n_blk, step, 0)

    return _body


@functools.partial(jax.jit, static_argnames=("m_blk", "depth", "interpret"))
def _graph_conv(x, adj, W, *, m_blk, depth, interpret=False):
    n, d_in = x.shape
    d_out = W.shape[0]
    return pl.pallas_call(
        _make_body(n, m_blk, depth),
        in_specs=[
            pl.BlockSpec(memory_space=pltpu.MemorySpace.HBM),
            pl.BlockSpec(memory_space=pltpu.MemorySpace.VMEM),
            pl.BlockSpec(memory_space=pltpu.MemorySpace.VMEM),
        ],
        out_specs=pl.BlockSpec(memory_space=pltpu.MemorySpace.VMEM),
        out_shape=jax.ShapeDtypeStruct((n, d_out), jnp.float32),
        scratch_shapes=[
            pltpu.VMEM((depth, m_blk, n), jnp.float32),
            pltpu.SemaphoreType.DMA((depth,)),
        ],
        compiler_params=pltpu.CompilerParams(
            vmem_limit_bytes=64 * 1024 * 1024),
        interpret=interpret,
    )(adj, x, W)


def kernel(x, adj, W):
    n = x.shape[0]
    m_blk = 200 if n % 200 == 0 else n
    depth = 4 if n // m_blk >= 4 else 1
    return _graph_conv(x, adj, W, m_blk=m_blk, depth=depth)
